# Initial kernel scaffold; baseline (speedup 1.0000x reference)
import jax
import jax.numpy as jnp
from jax.experimental import pallas as pl


def kernel(classifications, regressions, anchors, annotations, cur_state):
    raise NotImplementedError("write your pallas kernel here")


# speedup vs baseline: 1.4694x; 1.4694x over previous
import functools

import jax
import jax.numpy as jnp
from jax.experimental import pallas as pl

ALPHA = 0.25
A_TOTAL = 49104
BLK = 5456
NB = A_TOTAL // BLK
K = 24
C = 80


def _focal_kernel(cls_ref, reg_ref, anc_ref, ann_ref,
                  cls_out_ref, reg_out_ref, npos_out_ref):
    a_idx = pl.program_id(1)

    cls = jnp.clip(cls_ref[0], 0.0001, 1.0 - 0.0001)
    reg = reg_ref[0]
    anc = anc_ref[0]
    ann = ann_ref[0]

    ax1 = anc[:, 0]
    ay1 = anc[:, 1]
    ax2 = anc[:, 2]
    ay2 = anc[:, 3]
    aw = ax2 - ax1
    ah = ay2 - ay1
    acx = ax1 + 0.5 * aw
    acy = ay1 + 0.5 * ah
    area_a = aw * ah

    bx1 = ann[:, 0]
    by1 = ann[:, 1]
    bx2 = ann[:, 2]
    by2 = ann[:, 3]
    bcls = ann[:, 4]
    area_b = (bx2 - bx1) * (by2 - by1)

    iw = jnp.minimum(ax2[:, None], bx2[None, :]) - jnp.maximum(ax1[:, None], bx1[None, :])
    ih = jnp.minimum(ay2[:, None], by2[None, :]) - jnp.maximum(ay1[:, None], by1[None, :])
    iw = jnp.clip(iw, 0.0)
    ih = jnp.clip(ih, 0.0)
    inter = iw * ih
    ua = jnp.maximum(area_a[:, None] + area_b[None, :] - inter, 1e-8)
    iou = inter / ua

    iou_max = jnp.max(iou, axis=1)
    kidx = jax.lax.broadcasted_iota(jnp.int32, (BLK, K), 1)
    argmax = jnp.min(jnp.where(iou == iou_max[:, None], kidx, K), axis=1)
    sel = (kidx == argmax[:, None]).astype(cls.dtype)

    gx1 = jnp.sum(sel * bx1[None, :], axis=1)
    gy1 = jnp.sum(sel * by1[None, :], axis=1)
    gx2 = jnp.sum(sel * bx2[None, :], axis=1)
    gy2 = jnp.sum(sel * by2[None, :], axis=1)
    gcls = jnp.sum(sel * bcls[None, :], axis=1)

    pos = iou_max >= 0.5
    posf = pos.astype(cls.dtype)
    validf = jnp.logical_or(pos, iou_max < 0.4).astype(cls.dtype)

    one_m_cls = 1.0 - cls
    f0 = cls * cls * (-jnp.log(one_m_cls))
    s0 = jnp.sum(f0, axis=1)

    cidx = jax.lax.broadcasted_iota(jnp.int32, (BLK, C), 1)
    cmask = (cidx == gcls.astype(jnp.int32)[:, None]).astype(cls.dtype)
    pstar = jnp.sum(cmask * cls, axis=1)
    one_m_p = 1.0 - pstar
    f1p = one_m_p * one_m_p * (-jnp.log(pstar))
    f0p = pstar * pstar * (-jnp.log(one_m_p))
    cls_partial = ALPHA * jnp.sum(validf * s0 + posf * (f1p - f0p))
    npos_partial = jnp.sum(posf)

    gw = jnp.maximum(gx2 - gx1, 1.0)
    gh = jnp.maximum(gy2 - gy1, 1.0)
    gcx = gx1 + 0.5 * (gx2 - gx1)
    gcy = gy1 + 0.5 * (gy2 - gy1)
    dx = (gcx - acx) / aw * 10.0
    dy = (gcy - acy) / ah * 10.0
    dw = jnp.log(gw / aw) * 5.0
    dh = jnp.log(gh / ah) * 5.0

    def smooth_l1(t, r):
        diff = jnp.abs(t - r)
        return jnp.where(diff <= 1.0 / 9.0, 4.5 * diff * diff, diff - 0.5 / 9.0)

    rl = (smooth_l1(dx, reg[:, 0]) + smooth_l1(dy, reg[:, 1])
          + smooth_l1(dw, reg[:, 2]) + smooth_l1(dh, reg[:, 3]))
    reg_partial = jnp.sum(posf * rl)

    zero = jnp.zeros((1, 1, 1), jnp.float32)

    @pl.when(a_idx == 0)
    def _init():
        cls_out_ref[...] = zero
        reg_out_ref[...] = zero
        npos_out_ref[...] = zero

    cls_out_ref[...] += jnp.reshape(cls_partial, (1, 1, 1))
    reg_out_ref[...] += jnp.reshape(reg_partial, (1, 1, 1))
    npos_out_ref[...] += jnp.reshape(npos_partial, (1, 1, 1))


@jax.jit
def _run(classifications, regressions, anchors, annotations):
    B = classifications.shape[0]
    out_shape = jax.ShapeDtypeStruct((B, 1, 1), jnp.float32)
    cls_sum, reg_sum, npos = pl.pallas_call(
        _focal_kernel,
        grid=(B, NB),
        in_specs=[
            pl.BlockSpec((1, BLK, C), lambda b, a: (b, a, 0)),
            pl.BlockSpec((1, BLK, 4), lambda b, a: (b, a, 0)),
            pl.BlockSpec((1, BLK, 4), lambda b, a: (0, a, 0)),
            pl.BlockSpec((1, K, 5), lambda b, a: (b, 0, 0)),
        ],
        out_specs=[
            pl.BlockSpec((1, 1, 1), lambda b, a: (b, 0, 0)),
            pl.BlockSpec((1, 1, 1), lambda b, a: (b, 0, 0)),
            pl.BlockSpec((1, 1, 1), lambda b, a: (b, 0, 0)),
        ],
        out_shape=[out_shape, out_shape, out_shape],
    )(classifications, regressions, anchors, annotations)
    cls_sum = cls_sum[:, 0, 0]
    reg_sum = reg_sum[:, 0, 0]
    npos = npos[:, 0, 0]
    denom = jnp.maximum(npos, 1.0)
    cls_losses = cls_sum / denom
    reg_losses = jnp.where(npos > 0, reg_sum / (denom * 4.0), 0.0)
    return (jnp.mean(cls_losses, keepdims=True),
            jnp.mean(reg_losses, keepdims=True))


def kernel(classifications, regressions, anchors, annotations, cur_state):
    return _run(classifications, regressions, anchors, annotations)


# speedup vs baseline: 4.2229x; 2.8740x over previous
import jax
import jax.numpy as jnp
from jax.experimental import pallas as pl

ALPHA = 0.25
A_TOTAL = 49104
BLK = 6144
NB = 8
A_PAD = NB * BLK
K = 24
C = 80


def _focal_kernel(cls_ref, regt_ref, anct_ref, ann_ref,
                  cls_out_ref, reg_out_ref, npos_out_ref):
    a_idx = pl.program_id(1)

    cls = jnp.clip(cls_ref[0], 0.0001, 1.0 - 0.0001)
    regt = regt_ref[0][0:4]
    anct = anct_ref[0:4]
    ann = ann_ref[0]

    base = a_idx * BLK
    lane_ok = (jax.lax.broadcasted_iota(jnp.int32, (1, BLK), 1) + base
               < A_TOTAL)
    row_ok = (jax.lax.broadcasted_iota(jnp.int32, (BLK, 1), 0) + base
              < A_TOTAL)

    ax1 = anct[0:1]
    ay1 = anct[1:2]
    ax2 = anct[2:3]
    ay2 = anct[3:4]
    aw = ax2 - ax1
    ah = ay2 - ay1
    acx = ax1 + 0.5 * aw
    acy = ay1 + 0.5 * ah
    area_a = aw * ah

    bx1 = ann[:, 0:1]
    by1 = ann[:, 1:2]
    bx2 = ann[:, 2:3]
    by2 = ann[:, 3:4]
    bcls = ann[:, 4:5]
    area_b = (bx2 - bx1) * (by2 - by1)

    iw = jnp.minimum(ax2, bx2) - jnp.maximum(ax1, bx1)
    ih = jnp.minimum(ay2, by2) - jnp.maximum(ay1, by1)
    iw = jnp.clip(iw, 0.0)
    ih = jnp.clip(ih, 0.0)
    inter = iw * ih
    ua = jnp.maximum(area_a + area_b - inter, 1e-8)
    iou = inter / ua

    iou_max = jnp.max(iou, axis=0, keepdims=True)
    kidx = jax.lax.broadcasted_iota(jnp.int32, (K, BLK), 0)
    argmax = jnp.min(jnp.where(iou == iou_max, kidx, K), axis=0, keepdims=True)
    sel = (kidx == argmax).astype(jnp.float32)

    gx1 = jnp.sum(sel * bx1, axis=0, keepdims=True)
    gy1 = jnp.sum(sel * by1, axis=0, keepdims=True)
    gx2 = jnp.sum(sel * bx2, axis=0, keepdims=True)
    gy2 = jnp.sum(sel * by2, axis=0, keepdims=True)
    gcls = jnp.sum(sel * bcls, axis=0, keepdims=True)

    pos = jnp.logical_and(iou_max >= 0.5, lane_ok)
    posf = pos.astype(jnp.float32)
    npos_partial = jnp.sum(posf)

    gw = jnp.maximum(gx2 - gx1, 1.0)
    gh = jnp.maximum(gy2 - gy1, 1.0)
    gcx = gx1 + 0.5 * (gx2 - gx1)
    gcy = gy1 + 0.5 * (gy2 - gy1)
    dx = (gcx - acx) / aw * 10.0
    dy = (gcy - acy) / ah * 10.0
    dw = jnp.log(gw / aw) * 5.0
    dh = jnp.log(gh / ah) * 5.0
    t4 = jnp.concatenate([dx, dy, dw, dh], axis=0)
    diff = jnp.abs(t4 - regt)
    rl4 = jnp.where(diff <= 1.0 / 9.0, 4.5 * diff * diff, diff - 0.5 / 9.0)
    rl = jnp.sum(rl4, axis=0, keepdims=True)
    reg_partial = jnp.sum(jnp.where(pos, rl, 0.0))

    iou_max_t = jnp.transpose(iou_max)
    gcls_t = jnp.transpose(gcls).astype(jnp.int32)
    pos_t = iou_max_t >= 0.5
    valid_t = jnp.logical_and(
        jnp.logical_or(pos_t, iou_max_t < 0.4), row_ok)
    pos_t = jnp.logical_and(pos_t, row_ok)

    one_m_cls = 1.0 - cls
    f0 = cls * cls * (-jnp.log(one_m_cls))
    s0 = jnp.sum(f0, axis=1, keepdims=True)

    cidx = jax.lax.broadcasted_iota(jnp.int32, (BLK, C), 1)
    cmask = (cidx == gcls_t).astype(jnp.float32)
    pstar = jnp.sum(cmask * cls, axis=1, keepdims=True)
    one_m_p = 1.0 - pstar
    f1p = one_m_p * one_m_p * (-jnp.log(pstar))
    f0p = pstar * pstar * (-jnp.log(one_m_p))
    cls_partial = ALPHA * jnp.sum(jnp.where(valid_t, s0, 0.0)
                                  + jnp.where(pos_t, f1p - f0p, 0.0))

    zero = jnp.zeros((1, 1, 1), jnp.float32)

    @pl.when(a_idx == 0)
    def _init():
        cls_out_ref[...] = zero
        reg_out_ref[...] = zero
        npos_out_ref[...] = zero

    cls_out_ref[...] += jnp.reshape(cls_partial, (1, 1, 1))
    reg_out_ref[...] += jnp.reshape(reg_partial, (1, 1, 1))
    npos_out_ref[...] += jnp.reshape(npos_partial, (1, 1, 1))


@jax.jit
def _run(classifications, regressions, anchors, annotations):
    B = classifications.shape[0]
    A = classifications.shape[1]
    npad = A_PAD - A
    pad_box = jnp.tile(
        jnp.array([[0.0], [0.0], [64.0], [64.0]], jnp.float32), (1, npad))
    anct = jnp.concatenate(
        [jnp.concatenate([jnp.transpose(anchors[0]), pad_box], axis=1),
         jnp.zeros((4, A_PAD), jnp.float32)], axis=0)
    regt = jnp.concatenate(
        [jnp.transpose(regressions, (0, 2, 1)),
         jnp.zeros((B, 4, npad), jnp.float32)], axis=2)
    regt = jnp.concatenate(
        [regt, jnp.zeros((B, 4, A_PAD), jnp.float32)], axis=1)
    out_shape = jax.ShapeDtypeStruct((B, 1, 1), jnp.float32)
    cls_sum, reg_sum, npos = pl.pallas_call(
        _focal_kernel,
        grid=(B, NB),
        in_specs=[
            pl.BlockSpec((1, BLK, C), lambda b, a: (b, a, 0)),
            pl.BlockSpec((1, 8, BLK), lambda b, a: (b, 0, a)),
            pl.BlockSpec((8, BLK), lambda b, a: (0, a)),
            pl.BlockSpec((1, K, 5), lambda b, a: (b, 0, 0)),
        ],
        out_specs=[
            pl.BlockSpec((1, 1, 1), lambda b, a: (b, 0, 0)),
            pl.BlockSpec((1, 1, 1), lambda b, a: (b, 0, 0)),
            pl.BlockSpec((1, 1, 1), lambda b, a: (b, 0, 0)),
        ],
        out_shape=[out_shape, out_shape, out_shape],
    )(classifications, regt, anct, annotations)
    cls_sum = cls_sum[:, 0, 0]
    reg_sum = reg_sum[:, 0, 0]
    npos = npos[:, 0, 0]
    denom = jnp.maximum(npos, 1.0)
    cls_losses = cls_sum / denom
    reg_losses = jnp.where(npos > 0, reg_sum / (denom * 4.0), 0.0)
    return (jnp.mean(cls_losses, keepdims=True),
            jnp.mean(reg_losses, keepdims=True))


def kernel(classifications, regressions, anchors, annotations, cur_state):
    return _run(classifications, regressions, anchors, annotations)
